# Initial kernel scaffold; baseline (speedup 1.0000x reference)
import jax
import jax.numpy as jnp
from jax.experimental import pallas as pl


def kernel(x, conv1_w, conv1_b, conv2_w, conv2_b, W1, b1, W2, b2, fc_w, fc_b):
    raise NotImplementedError("write your pallas kernel here")


# speedup vs baseline: 7.7875x; 7.7875x over previous
import jax
import jax.numpy as jnp
from jax.experimental import pallas as pl
from jax.experimental.pallas import tpu as pltpu

_B = 128
_CIN = 16
_T = 2048
_T4 = 512
_HID = 256
_OUT = 64
_KNN = 8
_F = 32


def _conv_body(xt_ref, w1_ref, b1_ref, w2_ref, b2_ref, h_ref, mean_ref):
    g = pl.program_id(0)
    xg = xt_ref[...]
    z2 = jnp.zeros((2, 128), jnp.float32)
    xp = jnp.concatenate([z2, xg, z2], axis=0)
    acc = jnp.zeros((_T, 128), jnp.float32)
    for k in range(5):
        acc = acc + jax.lax.dot(xp[k:k + _T, :], w1_ref[k],
                                preferred_element_type=jnp.float32)
    h = jnp.maximum(acc + b1_ref[...], 0.0)
    h = jnp.max(h.reshape(_T // 2, 2, 128), axis=1)
    xp2 = jnp.concatenate([z2, h, z2], axis=0)
    acc2 = jnp.zeros((_T // 2, 256), jnp.float32)
    for k in range(5):
        acc2 = acc2 + jax.lax.dot(xp2[k:k + _T // 2, :], w2_ref[k],
                                  preferred_element_type=jnp.float32)
    h2 = jnp.maximum(acc2 + b2_ref[...], 0.0)
    h2 = jnp.max(h2.reshape(_T4, 2, 256), axis=1)
    h_ref[...] = h2.reshape(1, _T4, 256)
    part = jnp.sum(h2.reshape(_T4, 8, _F), axis=1) * (1.0 / _B)
    @pl.when(g == 0)
    def _():
        mean_ref[...] = part
    @pl.when(g > 0)
    def _():
        mean_ref[...] = mean_ref[...] + part


def _graph_body(mean_ref, m0_ref):
    mf = mean_ref[...]
    mm = mf * mf
    sq_col = jax.lax.dot(mm, jnp.ones((_F, 1), jnp.float32),
                         preferred_element_type=jnp.float32)
    sq_row = jax.lax.dot_general(jnp.ones((1, _F), jnp.float32), mm,
                                 (((1,), (1,)), ((), ())),
                                 preferred_element_type=jnp.float32)
    gram = jax.lax.dot_general(mf, mf, (((1,), (1,)), ((), ())),
                               preferred_element_type=jnp.float32)
    rows = jax.lax.broadcasted_iota(jnp.int32, (_T4, _T4), 0)
    cols = jax.lax.broadcasted_iota(jnp.int32, (_T4, _T4), 1)
    eye = rows == cols
    d = (sq_col + sq_row) - 2.0 * gram + jnp.where(eye, 1e9, 0.0)
    m0 = jnp.where(eye, 1.0, 0.0)
    for _ in range(_KNN):
        mn = jnp.min(d, axis=1, keepdims=True)
        idx = jnp.min(jnp.where(d == mn, cols, jnp.int32(1 << 30)),
                      axis=1, keepdims=True)
        sel = cols == idx
        m0 = m0 + jnp.where(sel, 1.0, 0.0)
        d = jnp.where(sel, jnp.float32(3e38), d)
    m0_ref[...] = m0 * (1.0 / 9.0)


def _gcn_body(hn_ref, w1_ref, b1_ref, w2_ref, b2_ref, fcw_ref, fcb_ref,
              out_ref):
    xb = hn_ref[...].reshape(8 * _T4, _F)
    h1 = jnp.maximum(jax.lax.dot(xb, w1_ref[...],
                                 preferred_element_type=jnp.float32)
                     + b1_ref[...], 0.0)
    h2 = jnp.maximum(jax.lax.dot(h1, w2_ref[...],
                                 preferred_element_type=jnp.float32)
                     + b2_ref[...], 0.0)
    pooled = jnp.sum(h2.reshape(8, _T4, _HID), axis=1) * (1.0 / _T4)
    out_ref[...] = jax.lax.dot(pooled, fcw_ref[...],
                               preferred_element_type=jnp.float32) + fcb_ref[...]


def _gcn0_body(hn_ref, m0_ref, w1_ref, b1_ref, w2_ref, b2_ref, fcw_ref,
               fcb_ref, out_ref):
    xb = hn_ref[...].reshape(8 * _T4, _F)
    m0 = m0_ref[...]
    xw = jax.lax.dot(xb, w1_ref[...], preferred_element_type=jnp.float32)
    z0 = jax.lax.dot(m0, xw[:_T4], preferred_element_type=jnp.float32)
    h1 = jnp.maximum(jnp.concatenate([z0, xw[_T4:]], axis=0) + b1_ref[...],
                     0.0)
    xw2 = jax.lax.dot(h1, w2_ref[...], preferred_element_type=jnp.float32)
    z02 = jax.lax.dot(m0, xw2[:_T4], preferred_element_type=jnp.float32)
    h2 = jnp.maximum(jnp.concatenate([z02, xw2[_T4:]], axis=0) + b2_ref[...],
                     0.0)
    pooled = jnp.sum(h2.reshape(8, _T4, _HID), axis=1) * (1.0 / _T4)
    out_ref[...] = jax.lax.dot(pooled, fcw_ref[...],
                               preferred_element_type=jnp.float32) + fcb_ref[...]


def kernel(x, conv1_w, conv1_b, conv2_w, conv2_b, W1, b1, W2, b2, fc_w, fc_b):
    f32 = jnp.float32
    xt = jnp.transpose(x, (2, 0, 1)).reshape(_T, _B * _CIN)
    eye8 = jnp.eye(8, dtype=f32)
    w1s = jnp.stack([jnp.kron(eye8, conv1_w[:, :, k].T) for k in range(5)])
    w2s = jnp.stack([jnp.kron(eye8, conv2_w[:, :, k].T) for k in range(5)])
    b1t = jnp.tile(conv1_b, 8).reshape(1, 128)
    b2t = jnp.tile(conv2_b, 8).reshape(1, 256)

    n_groups = _B * _CIN // 128
    h_all, mean_feat = pl.pallas_call(
        _conv_body,
        grid=(n_groups,),
        in_specs=[
            pl.BlockSpec((_T, 128), lambda g: (0, g)),
            pl.BlockSpec((5, 128, 128), lambda g: (0, 0, 0)),
            pl.BlockSpec((1, 128), lambda g: (0, 0)),
            pl.BlockSpec((5, 128, 256), lambda g: (0, 0, 0)),
            pl.BlockSpec((1, 256), lambda g: (0, 0)),
        ],
        out_specs=[
            pl.BlockSpec((1, _T4, 256), lambda g: (g, 0, 0)),
            pl.BlockSpec((_T4, _F), lambda g: (0, 0)),
        ],
        out_shape=[
            jax.ShapeDtypeStruct((n_groups, _T4, 256), f32),
            jax.ShapeDtypeStruct((_T4, _F), f32),
        ],
        compiler_params=pltpu.CompilerParams(
            dimension_semantics=("arbitrary",)),
    )(xt, w1s, b1t, w2s, b2t)

    m0 = pl.pallas_call(
        _graph_body,
        out_shape=jax.ShapeDtypeStruct((_T4, _T4), f32),
    )(mean_feat)

    hn_pb = h_all.reshape(n_groups, _T4, 8, _F).transpose(0, 2, 1, 3)
    hn_pb = hn_pb.reshape(_B, _T4, _F)

    b1r = b1.reshape(1, _HID)
    b2r = b2.reshape(1, _HID)
    fcbr = fc_b.reshape(1, _OUT)

    out_rest = pl.pallas_call(
        _gcn_body,
        grid=(15,),
        in_specs=[
            pl.BlockSpec((8, _T4, _F), lambda i: (i + 1, 0, 0)),
            pl.BlockSpec((_F, _HID), lambda i: (0, 0)),
            pl.BlockSpec((1, _HID), lambda i: (0, 0)),
            pl.BlockSpec((_HID, _HID), lambda i: (0, 0)),
            pl.BlockSpec((1, _HID), lambda i: (0, 0)),
            pl.BlockSpec((_HID, _OUT), lambda i: (0, 0)),
            pl.BlockSpec((1, _OUT), lambda i: (0, 0)),
        ],
        out_specs=pl.BlockSpec((8, _OUT), lambda i: (i, 0)),
        out_shape=jax.ShapeDtypeStruct((_B - 8, _OUT), f32),
    )(hn_pb, W1, b1r, W2, b2r, fc_w, fcbr)

    out_first = pl.pallas_call(
        _gcn0_body,
        out_shape=jax.ShapeDtypeStruct((8, _OUT), f32),
    )(hn_pb[:8], m0, W1, b1r, W2, b2r, fc_w, fcbr)

    return jnp.concatenate([out_first, out_rest], axis=0)


# speedup vs baseline: 8.5295x; 1.0953x over previous
import functools

import jax
import jax.numpy as jnp
from jax.experimental import pallas as pl
from jax.experimental.pallas import tpu as pltpu

_B = 128
_CIN = 16
_T = 2048
_T4 = 512
_HID = 256
_OUT = 64
_KNN = 8
_F = 32


def _conv_body(xt_ref, w1_ref, b1_ref, w2_ref, b2_ref, h_ref, d2_ref,
               mean_ref):
    g = pl.program_id(0)
    xg = xt_ref[...]
    z2 = jnp.zeros((2, 128), jnp.bfloat16)
    xp = jnp.concatenate([z2, xg, z2], axis=0)
    acc = jnp.zeros((_T, 128), jnp.float32)
    for k in range(5):
        acc = acc + jax.lax.dot(xp[k:k + _T, :], w1_ref[k],
                                preferred_element_type=jnp.float32)
    h = jnp.maximum(acc + b1_ref[...], 0.0)
    h = jnp.max(h.reshape(_T // 2, 2, 128), axis=1)
    hb = h.astype(jnp.bfloat16)
    xp2 = jnp.concatenate([z2, hb, z2], axis=0)
    acc2 = jnp.zeros((_T // 2, 256), jnp.float32)
    for k in range(5):
        acc2 = acc2 + jax.lax.dot(xp2[k:k + _T // 2, :], w2_ref[k],
                                  preferred_element_type=jnp.float32)
    h2 = jnp.maximum(acc2 + b2_ref[...], 0.0)
    h2 = jnp.max(h2.reshape(_T4, 2, 256), axis=1)
    h_ref[...] = h2.reshape(1, _T4, 256)
    part = jnp.sum(h2.reshape(_T4, 8, _F), axis=1) * (1.0 / _B)
    @pl.when(g == 0)
    def _():
        mean_ref[...] = part
    @pl.when(g > 0)
    def _():
        mean_ref[...] = mean_ref[...] + part

    @pl.when(g == pl.num_programs(0) - 1)
    def _():
        mf = mean_ref[...]
        mm = mf * mf
        sq_col = jax.lax.dot(mm, jnp.ones((_F, 1), jnp.float32),
                             preferred_element_type=jnp.float32)
        sq_row = jax.lax.dot_general(jnp.ones((1, _F), jnp.float32), mm,
                                     (((1,), (1,)), ((), ())),
                                     preferred_element_type=jnp.float32)
        gram = jax.lax.dot_general(mf, mf, (((1,), (1,)), ((), ())),
                                   preferred_element_type=jnp.float32)
        rows = jax.lax.broadcasted_iota(jnp.int32, (_T4, _T4), 0)
        cols = jax.lax.broadcasted_iota(jnp.int32, (_T4, _T4), 1)
        eye = rows == cols
        d2_ref[...] = (sq_col + sq_row) - 2.0 * gram + jnp.where(eye, 1e9, 0.0)


def _graph_body(d2_ref, m0_ref):
    cols = jax.lax.broadcasted_iota(jnp.int32, (_T4, _T4), 1)
    rows = jax.lax.broadcasted_iota(jnp.int32, (_T4, _T4), 0)
    d = d2_ref[...]
    m0 = jnp.where(rows == cols, 1.0, 0.0)
    for _ in range(_KNN):
        mn = jnp.min(d, axis=1, keepdims=True)
        idx = jnp.min(jnp.where(d == mn, cols, jnp.int32(1 << 30)),
                      axis=1, keepdims=True)
        sel = cols == idx
        m0 = m0 + jnp.where(sel, 1.0, 0.0)
        d = jnp.where(sel, jnp.float32(3e38), d)
    m0_ref[...] = m0 * (1.0 / 9.0)


def _gcn_batches(h2b, w1_ref, b1_ref, w2_ref, b2_ref, m0=None):
    pooled = []
    for bsub in range(8):
        xb = h2b[:, bsub * _F:(bsub + 1) * _F]
        xw = jax.lax.dot(xb, w1_ref[...], preferred_element_type=jnp.float32)
        if m0 is not None and bsub == 0:
            xw = jax.lax.dot(m0, xw, preferred_element_type=jnp.float32)
        h1 = jnp.maximum(xw + b1_ref[...], 0.0)
        xw2 = jax.lax.dot(h1.astype(jnp.bfloat16), w2_ref[...],
                          preferred_element_type=jnp.float32)
        if m0 is not None and bsub == 0:
            xw2 = jax.lax.dot(m0, xw2, preferred_element_type=jnp.float32)
        hg = jnp.maximum(xw2 + b2_ref[...], 0.0)
        pooled.append(jnp.sum(hg, axis=0, keepdims=True) * (1.0 / _T4))
    return jnp.concatenate(pooled, axis=0)


def _gcn_body(h_ref, w1_ref, b1_ref, w2_ref, b2_ref, fcw_ref, fcb_ref,
              out_ref):
    h2b = h_ref[0].astype(jnp.bfloat16)
    pooled = _gcn_batches(h2b, w1_ref, b1_ref, w2_ref, b2_ref)
    out_ref[...] = jax.lax.dot(pooled.astype(jnp.bfloat16), fcw_ref[...],
                               preferred_element_type=jnp.float32) + fcb_ref[...]


def _gcn0_body(h_ref, m0_ref, w1_ref, b1_ref, w2_ref, b2_ref, fcw_ref,
               fcb_ref, out_ref):
    h2b = h_ref[0].astype(jnp.bfloat16)
    pooled = _gcn_batches(h2b, w1_ref, b1_ref, w2_ref, b2_ref, m0=m0_ref[...])
    out_ref[...] = jax.lax.dot(pooled.astype(jnp.bfloat16), fcw_ref[...],
                               preferred_element_type=jnp.float32) + fcb_ref[...]


def kernel(x, conv1_w, conv1_b, conv2_w, conv2_b, W1, b1, W2, b2, fc_w, fc_b):
    f32 = jnp.float32
    bf16 = jnp.bfloat16
    xt = jnp.transpose(x, (2, 0, 1)).reshape(_T, _B * _CIN).astype(bf16)
    eye8 = jnp.eye(8, dtype=f32)
    w1s = jnp.stack([jnp.kron(eye8, conv1_w[:, :, k].T) for k in range(5)])
    w2s = jnp.stack([jnp.kron(eye8, conv2_w[:, :, k].T) for k in range(5)])
    w1s = w1s.astype(bf16)
    w2s = w2s.astype(bf16)
    b1t = jnp.tile(conv1_b, 8).reshape(1, 128)
    b2t = jnp.tile(conv2_b, 8).reshape(1, 256)

    n_groups = _B * _CIN // 128
    h_all, d2 = pl.pallas_call(
        _conv_body,
        grid=(n_groups,),
        in_specs=[
            pl.BlockSpec((_T, 128), lambda g: (0, g)),
            pl.BlockSpec((5, 128, 128), lambda g: (0, 0, 0)),
            pl.BlockSpec((1, 128), lambda g: (0, 0)),
            pl.BlockSpec((5, 128, 256), lambda g: (0, 0, 0)),
            pl.BlockSpec((1, 256), lambda g: (0, 0)),
        ],
        out_specs=[
            pl.BlockSpec((1, _T4, 256), lambda g: (g, 0, 0)),
            pl.BlockSpec((_T4, _T4), lambda g: (0, 0)),
        ],
        out_shape=[
            jax.ShapeDtypeStruct((n_groups, _T4, 256), f32),
            jax.ShapeDtypeStruct((_T4, _T4), f32),
        ],
        scratch_shapes=[pltpu.VMEM((_T4, _F), f32)],
        compiler_params=pltpu.CompilerParams(
            dimension_semantics=("arbitrary",)),
    )(xt, w1s, b1t, w2s, b2t)

    m0 = pl.pallas_call(
        _graph_body,
        out_shape=jax.ShapeDtypeStruct((_T4, _T4), f32),
    )(d2)

    W1b = W1.astype(bf16)
    W2b = W2.astype(bf16)
    fcwb = fc_w.astype(bf16)
    b1r = b1.reshape(1, _HID)
    b2r = b2.reshape(1, _HID)
    fcbr = fc_b.reshape(1, _OUT)

    out_rest = pl.pallas_call(
        _gcn_body,
        grid=(15,),
        in_specs=[
            pl.BlockSpec((1, _T4, 256), lambda i: (i + 1, 0, 0)),
            pl.BlockSpec((_F, _HID), lambda i: (0, 0)),
            pl.BlockSpec((1, _HID), lambda i: (0, 0)),
            pl.BlockSpec((_HID, _HID), lambda i: (0, 0)),
            pl.BlockSpec((1, _HID), lambda i: (0, 0)),
            pl.BlockSpec((_HID, _OUT), lambda i: (0, 0)),
            pl.BlockSpec((1, _OUT), lambda i: (0, 0)),
        ],
        out_specs=pl.BlockSpec((8, _OUT), lambda i: (i, 0)),
        out_shape=jax.ShapeDtypeStruct((_B - 8, _OUT), f32),
    )(h_all, W1b, b1r, W2b, b2r, fcwb, fcbr)

    out_first = pl.pallas_call(
        _gcn0_body,
        out_shape=jax.ShapeDtypeStruct((8, _OUT), f32),
    )(h_all[:1], m0, W1b, b1r, W2b, b2r, fcwb, fcbr)

    return jnp.concatenate([out_first, out_rest], axis=0)


# speedup vs baseline: 9.0297x; 1.0586x over previous
import functools

import jax
import jax.numpy as jnp
from jax.experimental import pallas as pl
from jax.experimental.pallas import tpu as pltpu

_B = 128
_CIN = 16
_T = 2048
_T4 = 512
_HID = 256
_OUT = 64
_KNN = 8
_F = 32


def _conv_body(xt_ref, w1_ref, b1_ref, w2_ref, b2_ref, h_ref, d2_ref,
               mean_ref):
    g = pl.program_id(0)
    xg = xt_ref[...]
    z2 = jnp.zeros((2, 128), jnp.float32)
    xp = jnp.concatenate([z2, xg, z2], axis=0)
    acc = jnp.zeros((_T, 128), jnp.float32)
    for k in range(5):
        acc = acc + jax.lax.dot(xp[k:k + _T, :], w1_ref[k],
                                preferred_element_type=jnp.float32)
    h = jnp.maximum(acc + b1_ref[...], 0.0)
    h = jnp.max(h.reshape(_T // 2, 2, 128), axis=1)
    xp2 = jnp.concatenate([z2, h, z2], axis=0)
    acc2 = jnp.zeros((_T // 2, 256), jnp.float32)
    for k in range(5):
        acc2 = acc2 + jax.lax.dot(xp2[k:k + _T // 2, :], w2_ref[k],
                                  preferred_element_type=jnp.float32)
    h2 = jnp.maximum(acc2 + b2_ref[...], 0.0)
    h2 = jnp.max(h2.reshape(_T4, 2, 256), axis=1)
    h_ref[...] = h2.reshape(1, _T4, 256)
    part = jnp.sum(h2.reshape(_T4, 8, _F), axis=1) * (1.0 / _B)
    @pl.when(g == 0)
    def _():
        mean_ref[...] = part
    @pl.when(g > 0)
    def _():
        mean_ref[...] = mean_ref[...] + part

    @pl.when(g == pl.num_programs(0) - 1)
    def _():
        mf = mean_ref[...]
        mm = mf * mf
        sq_col = jax.lax.dot(mm, jnp.ones((_F, 1), jnp.float32),
                             preferred_element_type=jnp.float32)
        sq_row = jax.lax.dot_general(jnp.ones((1, _F), jnp.float32), mm,
                                     (((1,), (1,)), ((), ())),
                                     preferred_element_type=jnp.float32)
        gram = jax.lax.dot_general(mf, mf, (((1,), (1,)), ((), ())),
                                   preferred_element_type=jnp.float32)
        rows = jax.lax.broadcasted_iota(jnp.int32, (_T4, _T4), 0)
        cols = jax.lax.broadcasted_iota(jnp.int32, (_T4, _T4), 1)
        eye = rows == cols
        d2_ref[...] = (sq_col + sq_row) - 2.0 * gram + jnp.where(eye, 1e9, 0.0)


def _graph_body(d2_ref, m0_ref):
    cols = jax.lax.broadcasted_iota(jnp.int32, (_T4, _T4), 1)
    rows = jax.lax.broadcasted_iota(jnp.int32, (_T4, _T4), 0)
    d = d2_ref[...]
    m0 = jnp.where(rows == cols, 1.0, 0.0)
    for _ in range(_KNN):
        mn = jnp.min(d, axis=1, keepdims=True)
        idx = jnp.min(jnp.where(d == mn, cols, jnp.int32(1 << 30)),
                      axis=1, keepdims=True)
        sel = cols == idx
        m0 = m0 + jnp.where(sel, 1.0, 0.0)
        d = jnp.where(sel, jnp.float32(3e38), d)
    m0_ref[...] = m0 * (1.0 / 9.0)


def _gcn_batches(h2b, w1_ref, b1_ref, w2_ref, b2_ref, m0=None):
    pooled = []
    for bsub in range(8):
        xb = h2b[:, bsub * _F:(bsub + 1) * _F]
        xw = jax.lax.dot(xb, w1_ref[...], preferred_element_type=jnp.float32)
        if m0 is not None and bsub == 0:
            xw = jax.lax.dot(m0, xw, preferred_element_type=jnp.float32)
        h1 = jnp.maximum(xw + b1_ref[...], 0.0)
        xw2 = jax.lax.dot(h1, w2_ref[...],
                          preferred_element_type=jnp.float32)
        if m0 is not None and bsub == 0:
            xw2 = jax.lax.dot(m0, xw2, preferred_element_type=jnp.float32)
        hg = jnp.maximum(xw2 + b2_ref[...], 0.0)
        pooled.append(jnp.sum(hg, axis=0, keepdims=True) * (1.0 / _T4))
    return jnp.concatenate(pooled, axis=0)


def _gcn_body(h_ref, w1_ref, b1_ref, w2_ref, b2_ref, fcw_ref, fcb_ref,
              out_ref):
    h2b = h_ref[0]
    pooled = _gcn_batches(h2b, w1_ref, b1_ref, w2_ref, b2_ref)
    out_ref[...] = jax.lax.dot(pooled, fcw_ref[...],
                               preferred_element_type=jnp.float32) + fcb_ref[...]


def _gcn0_body(h_ref, m0_ref, w1_ref, b1_ref, w2_ref, b2_ref, fcw_ref,
               fcb_ref, out_ref):
    h2b = h_ref[0]
    pooled = _gcn_batches(h2b, w1_ref, b1_ref, w2_ref, b2_ref, m0=m0_ref[...])
    out_ref[...] = jax.lax.dot(pooled, fcw_ref[...],
                               preferred_element_type=jnp.float32) + fcb_ref[...]


def kernel(x, conv1_w, conv1_b, conv2_w, conv2_b, W1, b1, W2, b2, fc_w, fc_b):
    f32 = jnp.float32
    xt = jnp.transpose(x, (2, 0, 1)).reshape(_T, _B * _CIN)
    eye8 = jnp.eye(8, dtype=f32)
    w1s = jnp.stack([jnp.kron(eye8, conv1_w[:, :, k].T) for k in range(5)])
    w2s = jnp.stack([jnp.kron(eye8, conv2_w[:, :, k].T) for k in range(5)])
    b1t = jnp.tile(conv1_b, 8).reshape(1, 128)
    b2t = jnp.tile(conv2_b, 8).reshape(1, 256)

    n_groups = _B * _CIN // 128
    h_all, d2 = pl.pallas_call(
        _conv_body,
        grid=(n_groups,),
        in_specs=[
            pl.BlockSpec((_T, 128), lambda g: (0, g)),
            pl.BlockSpec((5, 128, 128), lambda g: (0, 0, 0)),
            pl.BlockSpec((1, 128), lambda g: (0, 0)),
            pl.BlockSpec((5, 128, 256), lambda g: (0, 0, 0)),
            pl.BlockSpec((1, 256), lambda g: (0, 0)),
        ],
        out_specs=[
            pl.BlockSpec((1, _T4, 256), lambda g: (g, 0, 0)),
            pl.BlockSpec((_T4, _T4), lambda g: (0, 0)),
        ],
        out_shape=[
            jax.ShapeDtypeStruct((n_groups, _T4, 256), f32),
            jax.ShapeDtypeStruct((_T4, _T4), f32),
        ],
        scratch_shapes=[pltpu.VMEM((_T4, _F), f32)],
        compiler_params=pltpu.CompilerParams(
            dimension_semantics=("arbitrary",)),
    )(xt, w1s, b1t, w2s, b2t)

    m0 = pl.pallas_call(
        _graph_body,
        out_shape=jax.ShapeDtypeStruct((_T4, _T4), f32),
    )(d2)

    W1b = W1
    W2b = W2
    fcwb = fc_w
    b1r = b1.reshape(1, _HID)
    b2r = b2.reshape(1, _HID)
    fcbr = fc_b.reshape(1, _OUT)

    out_rest = pl.pallas_call(
        _gcn_body,
        grid=(15,),
        in_specs=[
            pl.BlockSpec((1, _T4, 256), lambda i: (i + 1, 0, 0)),
            pl.BlockSpec((_F, _HID), lambda i: (0, 0)),
            pl.BlockSpec((1, _HID), lambda i: (0, 0)),
            pl.BlockSpec((_HID, _HID), lambda i: (0, 0)),
            pl.BlockSpec((1, _HID), lambda i: (0, 0)),
            pl.BlockSpec((_HID, _OUT), lambda i: (0, 0)),
            pl.BlockSpec((1, _OUT), lambda i: (0, 0)),
        ],
        out_specs=pl.BlockSpec((8, _OUT), lambda i: (i, 0)),
        out_shape=jax.ShapeDtypeStruct((_B - 8, _OUT), f32),
    )(h_all, W1b, b1r, W2b, b2r, fcwb, fcbr)

    out_first = pl.pallas_call(
        _gcn0_body,
        out_shape=jax.ShapeDtypeStruct((8, _OUT), f32),
    )(h_all[:1], m0, W1b, b1r, W2b, b2r, fcwb, fcbr)

    return jnp.concatenate([out_first, out_rest], axis=0)
